# Initial kernel scaffold; baseline (speedup 1.0000x reference)
import jax
import jax.numpy as jnp
from jax.experimental import pallas as pl


def kernel(three_class_prob, gt_betti_numbers):
    raise NotImplementedError("write your pallas kernel here")


# speedup vs baseline: 9.3079x; 9.3079x over previous
import jax
import jax.numpy as jnp
from jax.experimental import pallas as pl

REGION = 32
GRID_R = 16
H = W = 512
RATIO = 0.25


def _betti_count_kernel(x_ref, reg_ref, img_ref):
    col = jax.lax.broadcasted_iota(jnp.int32, (H, W), 1)
    row = jax.lax.broadcasted_iota(jnp.int32, (H, W), 0)
    wmask = (col != W - 1).astype(jnp.float32)
    hmask = (row != H - 1).astype(jnp.float32)
    cmask = (col % REGION != REGION - 1).astype(jnp.float32)
    rmask = (row % REGION != REGION - 1).astype(jnp.float32)
    rcmask = rmask * cmask
    p_col = (jax.lax.broadcasted_iota(jnp.int32, (W, GRID_R), 0) // REGION ==
             jax.lax.broadcasted_iota(jnp.int32, (W, GRID_R), 1)
             ).astype(jnp.bfloat16)

    def one_channel(b):
        bR = jnp.roll(b, -1, axis=1)
        bD = jnp.roll(b, -1, axis=0)
        eh = b * bR * wmask
        ev = b * bD * hmask
        fq = eh * jnp.roll(eh, -1, axis=0) * hmask
        chi_full = b - eh - ev + fq
        chi_reg = b - eh * cmask - ev * rmask + fq * rcmask
        rp_full = jnp.sum(chi_full.reshape(GRID_R, REGION, W), axis=1)
        rp_reg = jnp.sum(chi_reg.reshape(GRID_R, REGION, W), axis=1)
        chi_img = jnp.sum(rp_full)
        pool = jax.lax.dot_general(
            rp_reg.astype(jnp.bfloat16), p_col,
            (((1,), (0,)), ((), ())), preferred_element_type=jnp.float32)
        return pool, chi_img

    pool0, chi0 = one_channel((x_ref[0, 0] > 0.5).astype(jnp.float32))
    pool1, chi1 = one_channel((x_ref[0, 1] > 0.5).astype(jnp.float32))
    reg_ref[0, 0] = pool0
    reg_ref[0, 1] = pool1
    lane = jax.lax.broadcasted_iota(jnp.int32, (1, 1, 8), 2)
    img_ref[...] = jnp.where(lane == 0, chi0,
                             jnp.where(lane == 1, chi1, 0.0))


def _select_mask_kernel(reg_ref, img_ref, gt_ref, out_ref):
    n = pl.program_id(0)
    B = reg_ref.shape[0]
    g = gt_ref[:, 0, :]
    ci = img_ref[:, 0, :]

    def six_err(b0a, b1a, b0b, b1b, g0, g1, g2, g3, g4, g5):
        return (jnp.abs(b0a - g0) + jnp.abs(b1a - g1)
                + jnp.abs(b0b - g2) + jnp.abs(b1b - g3)
                + jnp.abs(b0a - g4) + jnp.abs(b1a - g5))

    chi0 = ci[:, 0:1]
    chi1 = ci[:, 1:2]
    topo = six_err(jnp.maximum(chi0, 0.0), jnp.maximum(-chi0, 0.0),
                   jnp.maximum(chi1, 0.0), jnp.maximum(-chi1, 0.0),
                   g[:, 0:1], g[:, 1:2], g[:, 2:3], g[:, 3:4],
                   g[:, 4:5], g[:, 5:6])
    mean_i = jnp.sum(topo) / B
    var_i = jnp.sum((topo - mean_i) ** 2) / (B - 1)
    thr_i = mean_i + RATIO * jnp.sqrt(var_i)
    iota_b = jax.lax.broadcasted_iota(jnp.int32, (B, 1), 0)
    own_topo = jnp.sum(jnp.where(iota_b == n, topo, 0.0))
    sel_img = own_topo > thr_i

    cr = reg_ref[...]
    c0 = cr[:, 0]
    c1 = cr[:, 1]

    def gk(k):
        return g[:, k:k + 1][:, :, None]

    rerr = six_err(jnp.maximum(c0, 0.0), jnp.maximum(-c0, 0.0),
                   jnp.maximum(c1, 0.0), jnp.maximum(-c1, 0.0),
                   gk(0), gk(1), gk(2), gk(3), gk(4), gk(5))
    nreg = B * GRID_R * GRID_R
    mean_r = jnp.sum(rerr) / nreg
    var_r = jnp.sum((rerr - mean_r) ** 2) / (nreg - 1)
    thr_r = mean_r + RATIO * jnp.sqrt(var_r)
    iota_b3 = jax.lax.broadcasted_iota(jnp.int32, (B, GRID_R, GRID_R), 0)
    own_rerr = jnp.sum(jnp.where(iota_b3 == n, rerr, 0.0), axis=0)
    sel = jnp.logical_and(own_rerr > thr_r, sel_img)

    s16 = sel.astype(jnp.bfloat16)
    qt = (jax.lax.broadcasted_iota(jnp.int32, (H, GRID_R), 0) // REGION ==
          jax.lax.broadcasted_iota(jnp.int32, (H, GRID_R), 1)
          ).astype(jnp.bfloat16)
    q = (jax.lax.broadcasted_iota(jnp.int32, (GRID_R, W), 1) // REGION ==
         jax.lax.broadcasted_iota(jnp.int32, (GRID_R, W), 0)
         ).astype(jnp.bfloat16)
    t1 = jax.lax.dot_general(qt, s16, (((1,), (0,)), ((), ())),
                             preferred_element_type=jnp.float32)
    m = jax.lax.dot_general(t1.astype(jnp.bfloat16), q,
                            (((1,), (0,)), ((), ())),
                            preferred_element_type=jnp.float32)
    mask = m > 0.5
    out_ref[0, 0] = mask
    out_ref[0, 1] = mask
    out_ref[0, 2] = mask


def kernel(three_class_prob, gt_betti_numbers):
    B = three_class_prob.shape[0]
    x2 = three_class_prob[:, :2]
    gt8 = jnp.concatenate(
        [gt_betti_numbers.reshape(B, 6).astype(jnp.float32),
         jnp.zeros((B, 2), jnp.float32)], axis=1).reshape(B, 1, 8)

    chi_reg, chi_img = pl.pallas_call(
        _betti_count_kernel,
        grid=(B,),
        in_specs=[pl.BlockSpec((1, 2, H, W), lambda n: (n, 0, 0, 0))],
        out_specs=[
            pl.BlockSpec((1, 2, GRID_R, GRID_R), lambda n: (n, 0, 0, 0)),
            pl.BlockSpec((1, 1, 8), lambda n: (n, 0, 0)),
        ],
        out_shape=[
            jax.ShapeDtypeStruct((B, 2, GRID_R, GRID_R), jnp.float32),
            jax.ShapeDtypeStruct((B, 1, 8), jnp.float32),
        ],
        interpret=False,
    )(x2)

    masks = pl.pallas_call(
        _select_mask_kernel,
        grid=(B,),
        in_specs=[
            pl.BlockSpec((B, 2, GRID_R, GRID_R), lambda n: (0, 0, 0, 0)),
            pl.BlockSpec((B, 1, 8), lambda n: (0, 0, 0)),
            pl.BlockSpec((B, 1, 8), lambda n: (0, 0, 0)),
        ],
        out_specs=pl.BlockSpec((1, 3, H, W), lambda n: (n, 0, 0, 0)),
        out_shape=jax.ShapeDtypeStruct((B, 3, H, W), jnp.bool_),
        interpret=False,
    )(chi_reg, chi_img, gt8)
    return masks


# speedup vs baseline: 13.7051x; 1.4724x over previous
import jax
import jax.numpy as jnp
from jax.experimental import pallas as pl
from jax.experimental.pallas import tpu as pltpu

REGION = 32
GRID_R = 16
H = W = 512
RATIO = 0.25


def _betti_count_kernel(x0_ref, x1_ref, reg_ref, img_ref):
    col1 = jax.lax.broadcasted_iota(jnp.int32, (1, W), 1)
    cbar = (col1 % REGION == REGION - 1).astype(jnp.float32)
    wbar = (col1 == W - 1).astype(jnp.float32)
    wm = 1.0 - wbar
    cm = 1.0 - cbar
    p_col = (jax.lax.broadcasted_iota(jnp.int32, (W, GRID_R), 0) // REGION ==
             jax.lax.broadcasted_iota(jnp.int32, (W, GRID_R), 1)
             ).astype(jnp.bfloat16)

    def one_channel(x):
        b = (x > 0.5).astype(jnp.float32)
        bR = jnp.roll(b, -1, axis=1)
        bD = jnp.roll(b, -1, axis=0)
        mh = b * bR
        mv = b * bD
        mf = mh * jnp.roll(mh, -1, axis=0)
        chi_raw = b - mh - mv + mf
        d = mh - mf
        rp_chi = jnp.sum(chi_raw.reshape(GRID_R, REGION, W), axis=1)
        rp_d = jnp.sum(d.reshape(GRID_R, REGION, W), axis=1)
        mvb = mv.reshape(GRID_R, REGION, W)[:, REGION - 1, :]
        mfb = mf.reshape(GRID_R, REGION, W)[:, REGION - 1, :]
        rp_reg = rp_chi + rp_d * cbar + mvb - mfb * cm
        pool = jax.lax.dot_general(
            rp_reg.astype(jnp.bfloat16), p_col,
            (((1,), (0,)), ((), ())), preferred_element_type=jnp.float32)
        chi_img = (jnp.sum(rp_chi) + jnp.sum(rp_d * wbar)
                   + jnp.sum(mvb[GRID_R - 1:, :])
                   - jnp.sum(mfb[GRID_R - 1:, :] * wm))
        return pool, chi_img

    pool0, chi0 = one_channel(x0_ref[0, 0])
    pool1, chi1 = one_channel(x1_ref[0, 0])
    reg_ref[0, 0] = pool0
    reg_ref[0, 1] = pool1
    lane = jax.lax.broadcasted_iota(jnp.int32, (1, 1, 8), 2)
    img_ref[...] = jnp.where(lane == 0, chi0,
                             jnp.where(lane == 1, chi1, 0.0))


def _select_mask_kernel(reg_ref, img_ref, gt_ref, out_ref, sel_ref):
    n = pl.program_id(0)
    B = reg_ref.shape[0]

    @pl.when(n == 0)
    def _compute_selection():
        g = gt_ref[:, 0, :]
        ci = img_ref[:, 0, :]

        def six_err(b0a, b1a, b0b, b1b, g0, g1, g2, g3, g4, g5):
            return (jnp.abs(b0a - g0) + jnp.abs(b1a - g1)
                    + jnp.abs(b0b - g2) + jnp.abs(b1b - g3)
                    + jnp.abs(b0a - g4) + jnp.abs(b1a - g5))

        chi0 = ci[:, 0:1]
        chi1 = ci[:, 1:2]
        topo = six_err(jnp.maximum(chi0, 0.0), jnp.maximum(-chi0, 0.0),
                       jnp.maximum(chi1, 0.0), jnp.maximum(-chi1, 0.0),
                       g[:, 0:1], g[:, 1:2], g[:, 2:3], g[:, 3:4],
                       g[:, 4:5], g[:, 5:6])
        mean_i = jnp.sum(topo) / B
        var_i = jnp.sum((topo - mean_i) ** 2) / (B - 1)
        thr_i = mean_i + RATIO * jnp.sqrt(var_i)

        cr = reg_ref[...]
        c0 = cr[:, 0]
        c1 = cr[:, 1]

        def gk(k):
            return g[:, k:k + 1][:, :, None]

        rerr = six_err(jnp.maximum(c0, 0.0), jnp.maximum(-c0, 0.0),
                       jnp.maximum(c1, 0.0), jnp.maximum(-c1, 0.0),
                       gk(0), gk(1), gk(2), gk(3), gk(4), gk(5))
        nreg = B * GRID_R * GRID_R
        mean_r = jnp.sum(rerr) / nreg
        var_r = jnp.sum((rerr - mean_r) ** 2) / (nreg - 1)
        thr_r = mean_r + RATIO * jnp.sqrt(var_r)

        sel = jnp.logical_and(rerr > thr_r, topo[:, :, None] > thr_i)
        sel_ref[...] = sel.astype(jnp.float32)

    s16 = sel_ref[n].astype(jnp.bfloat16)
    qt = (jax.lax.broadcasted_iota(jnp.int32, (H, GRID_R), 0) // REGION ==
          jax.lax.broadcasted_iota(jnp.int32, (H, GRID_R), 1)
          ).astype(jnp.bfloat16)
    q = (jax.lax.broadcasted_iota(jnp.int32, (GRID_R, W), 1) // REGION ==
         jax.lax.broadcasted_iota(jnp.int32, (GRID_R, W), 0)
         ).astype(jnp.bfloat16)
    t1 = jax.lax.dot_general(qt, s16, (((1,), (0,)), ((), ())),
                             preferred_element_type=jnp.float32)
    m = jax.lax.dot_general(t1.astype(jnp.bfloat16), q,
                            (((1,), (0,)), ((), ())),
                            preferred_element_type=jnp.float32)
    mask = m > 0.5
    out_ref[0, 0] = mask
    out_ref[0, 1] = mask
    out_ref[0, 2] = mask


def kernel(three_class_prob, gt_betti_numbers):
    B = three_class_prob.shape[0]
    gt8 = jnp.concatenate(
        [gt_betti_numbers.reshape(B, 6).astype(jnp.float32),
         jnp.zeros((B, 2), jnp.float32)], axis=1).reshape(B, 1, 8)

    chi_reg, chi_img = pl.pallas_call(
        _betti_count_kernel,
        grid=(B,),
        in_specs=[
            pl.BlockSpec((1, 1, H, W), lambda n: (n, 0, 0, 0)),
            pl.BlockSpec((1, 1, H, W), lambda n: (n, 1, 0, 0)),
        ],
        out_specs=[
            pl.BlockSpec((1, 2, GRID_R, GRID_R), lambda n: (n, 0, 0, 0)),
            pl.BlockSpec((1, 1, 8), lambda n: (n, 0, 0)),
        ],
        out_shape=[
            jax.ShapeDtypeStruct((B, 2, GRID_R, GRID_R), jnp.float32),
            jax.ShapeDtypeStruct((B, 1, 8), jnp.float32),
        ],
        interpret=False,
    )(three_class_prob, three_class_prob)

    masks = pl.pallas_call(
        _select_mask_kernel,
        grid=(B,),
        in_specs=[
            pl.BlockSpec((B, 2, GRID_R, GRID_R), lambda n: (0, 0, 0, 0)),
            pl.BlockSpec((B, 1, 8), lambda n: (0, 0, 0)),
            pl.BlockSpec((B, 1, 8), lambda n: (0, 0, 0)),
        ],
        out_specs=pl.BlockSpec((1, 3, H, W), lambda n: (n, 0, 0, 0)),
        out_shape=jax.ShapeDtypeStruct((B, 3, H, W), jnp.bool_),
        scratch_shapes=[pltpu.VMEM((B, GRID_R, GRID_R), jnp.float32)],
        interpret=False,
    )(chi_reg, chi_img, gt8)
    return masks


# speedup vs baseline: 13.7062x; 1.0001x over previous
import jax
import jax.numpy as jnp
from jax.experimental import pallas as pl
from jax.experimental.pallas import tpu as pltpu

REGION = 32
GRID_R = 16
H = W = 512
RATIO = 0.25


def _betti_count_kernel(x0_ref, x1_ref, reg_ref, img_ref):
    col1 = jax.lax.broadcasted_iota(jnp.int32, (1, W), 1)
    cbar = (col1 % REGION == REGION - 1).astype(jnp.float32)
    wbar = (col1 == W - 1).astype(jnp.float32)
    wm = 1.0 - wbar
    cm = 1.0 - cbar
    p_col = (jax.lax.broadcasted_iota(jnp.int32, (W, GRID_R), 0) // REGION ==
             jax.lax.broadcasted_iota(jnp.int32, (W, GRID_R), 1)
             ).astype(jnp.bfloat16)

    def one_channel(x):
        b = (x > 0.5).astype(jnp.float32)
        bR = jnp.roll(b, -1, axis=1)
        bD = jnp.roll(b, -1, axis=0)
        mh = b * bR
        mv = b * bD
        mf = mh * jnp.roll(mh, -1, axis=0)
        chi_raw = b - mh - mv + mf
        d = mh - mf
        rp_chi = jnp.sum(chi_raw.reshape(GRID_R, REGION, W), axis=1)
        rp_d = jnp.sum(d.reshape(GRID_R, REGION, W), axis=1)
        mvb = mv.reshape(GRID_R, REGION, W)[:, REGION - 1, :]
        mfb = mf.reshape(GRID_R, REGION, W)[:, REGION - 1, :]
        rp_reg = rp_chi + rp_d * cbar + mvb - mfb * cm
        pool = jax.lax.dot_general(
            rp_reg.astype(jnp.bfloat16), p_col,
            (((1,), (0,)), ((), ())), preferred_element_type=jnp.float32)
        chi_img = (jnp.sum(rp_chi) + jnp.sum(rp_d * wbar)
                   + jnp.sum(mvb[GRID_R - 1:, :])
                   - jnp.sum(mfb[GRID_R - 1:, :] * wm))
        return pool, chi_img

    pool0, chi0 = one_channel(x0_ref[0, 0])
    pool1, chi1 = one_channel(x1_ref[0, 0])
    reg_ref[0, 0] = pool0
    reg_ref[0, 1] = pool1
    lane = jax.lax.broadcasted_iota(jnp.int32, (1, 1, 8), 2)
    img_ref[...] = jnp.where(lane == 0, chi0,
                             jnp.where(lane == 1, chi1, 0.0))


def _select_mask_kernel(reg_ref, img_ref, gt_ref, out_ref, sel_ref):
    n = pl.program_id(0)
    B = reg_ref.shape[0]

    @pl.when(n == 0)
    def _compute_selection():
        g = gt_ref[:, 0, :]
        ci = img_ref[:, 0, :]

        def six_err(b0a, b1a, b0b, b1b, g0, g1, g2, g3, g4, g5):
            return (jnp.abs(b0a - g0) + jnp.abs(b1a - g1)
                    + jnp.abs(b0b - g2) + jnp.abs(b1b - g3)
                    + jnp.abs(b0a - g4) + jnp.abs(b1a - g5))

        chi0 = ci[:, 0:1]
        chi1 = ci[:, 1:2]
        topo = six_err(jnp.maximum(chi0, 0.0), jnp.maximum(-chi0, 0.0),
                       jnp.maximum(chi1, 0.0), jnp.maximum(-chi1, 0.0),
                       g[:, 0:1], g[:, 1:2], g[:, 2:3], g[:, 3:4],
                       g[:, 4:5], g[:, 5:6])
        mean_i = jnp.sum(topo) / B
        var_i = jnp.sum((topo - mean_i) ** 2) / (B - 1)
        thr_i = mean_i + RATIO * jnp.sqrt(var_i)

        cr = reg_ref[...]
        c0 = cr[:, 0]
        c1 = cr[:, 1]

        def gk(k):
            return g[:, k:k + 1][:, :, None]

        rerr = six_err(jnp.maximum(c0, 0.0), jnp.maximum(-c0, 0.0),
                       jnp.maximum(c1, 0.0), jnp.maximum(-c1, 0.0),
                       gk(0), gk(1), gk(2), gk(3), gk(4), gk(5))
        nreg = B * GRID_R * GRID_R
        mean_r = jnp.sum(rerr) / nreg
        var_r = jnp.sum((rerr - mean_r) ** 2) / (nreg - 1)
        thr_r = mean_r + RATIO * jnp.sqrt(var_r)

        sel = jnp.logical_and(rerr > thr_r, topo[:, :, None] > thr_i)
        sel_ref[...] = sel.astype(jnp.float32)

    s16 = sel_ref[n].astype(jnp.bfloat16)
    qt = (jax.lax.broadcasted_iota(jnp.int32, (H, GRID_R), 0) // REGION ==
          jax.lax.broadcasted_iota(jnp.int32, (H, GRID_R), 1)
          ).astype(jnp.bfloat16)
    q = (jax.lax.broadcasted_iota(jnp.int32, (GRID_R, W), 1) // REGION ==
         jax.lax.broadcasted_iota(jnp.int32, (GRID_R, W), 0)
         ).astype(jnp.bfloat16)
    t1 = jax.lax.dot_general(qt, s16, (((1,), (0,)), ((), ())),
                             preferred_element_type=jnp.float32)
    m = jax.lax.dot_general(t1.astype(jnp.bfloat16), q,
                            (((1,), (0,)), ((), ())),
                            preferred_element_type=jnp.float32)
    mask = m > 0.5
    out_ref[0, 0] = mask
    out_ref[0, 1] = mask
    out_ref[0, 2] = mask


def kernel(three_class_prob, gt_betti_numbers):
    B = three_class_prob.shape[0]
    gt8 = jnp.concatenate(
        [gt_betti_numbers.reshape(B, 6).astype(jnp.float32),
         jnp.zeros((B, 2), jnp.float32)], axis=1).reshape(B, 1, 8)

    chi_reg, chi_img = pl.pallas_call(
        _betti_count_kernel,
        grid=(B,),
        in_specs=[
            pl.BlockSpec((1, 1, H, W), lambda n: (n, 0, 0, 0)),
            pl.BlockSpec((1, 1, H, W), lambda n: (n, 1, 0, 0)),
        ],
        out_specs=[
            pl.BlockSpec((1, 2, GRID_R, GRID_R), lambda n: (n, 0, 0, 0)),
            pl.BlockSpec((1, 1, 8), lambda n: (n, 0, 0)),
        ],
        out_shape=[
            jax.ShapeDtypeStruct((B, 2, GRID_R, GRID_R), jnp.float32),
            jax.ShapeDtypeStruct((B, 1, 8), jnp.float32),
        ],
        compiler_params=pltpu.CompilerParams(
            dimension_semantics=("parallel",)),
        interpret=False,
    )(three_class_prob, three_class_prob)

    masks = pl.pallas_call(
        _select_mask_kernel,
        grid=(B,),
        in_specs=[
            pl.BlockSpec((B, 2, GRID_R, GRID_R), lambda n: (0, 0, 0, 0)),
            pl.BlockSpec((B, 1, 8), lambda n: (0, 0, 0)),
            pl.BlockSpec((B, 1, 8), lambda n: (0, 0, 0)),
        ],
        out_specs=pl.BlockSpec((1, 3, H, W), lambda n: (n, 0, 0, 0)),
        out_shape=jax.ShapeDtypeStruct((B, 3, H, W), jnp.bool_),
        scratch_shapes=[pltpu.VMEM((B, GRID_R, GRID_R), jnp.float32)],
        interpret=False,
    )(chi_reg, chi_img, gt8)
    return masks
